# baseline (speedup 1.0000x reference)
import jax
import jax.numpy as jnp
from jax.experimental import pallas as pl
from jax.experimental.pallas import tpu as pltpu

B = 32
H = 768
I = 768
E = 16
NSLOT = 2
ALPHA = 1.702
LIMIT = 7.0
EPS = 1e-5
NEG_BIG = -1e30


def _copies(w1_hbm, w2_hbm, w1_buf, w2_buf, sems, e, s):
    return (
        pltpu.make_async_copy(
            w1_hbm.at[pl.ds(e, 1), pl.ds(0, I), :],
            w1_buf.at[pl.ds(s, 1), pl.ds(0, I), :], sems.at[0, s]),
        pltpu.make_async_copy(
            w1_hbm.at[pl.ds(e, 1), pl.ds(I, I), :],
            w1_buf.at[pl.ds(s, 1), pl.ds(I, I), :], sems.at[1, s]),
        pltpu.make_async_copy(
            w2_hbm.at[pl.ds(e, 1)],
            w2_buf.at[pl.ds(s, 1)], sems.at[2, s]),
    )


def _ffn_half(t16, w1h, b1_ref, psel_ref):
    t1 = jax.lax.dot_general(
        t16, w1h.astype(jnp.bfloat16), (((1,), (1,)), ((), ())),
        preferred_element_type=jnp.float32) + b1_ref[...]
    glu = jnp.minimum(t1, LIMIT)
    og = glu * jax.nn.sigmoid(ALPHA * glu)
    lin = jnp.clip(t1, -LIMIT, LIMIT) + 1.0
    lin_next = pltpu.roll(lin, I - 1, 1)
    h_full = (og * lin_next).astype(jnp.bfloat16)
    return jax.lax.dot_general(
        h_full, psel_ref[...], (((1,), (0,)), ((), ())),
        preferred_element_type=jnp.float32).astype(jnp.bfloat16)


def _moe_kernel(x_ref, scale_ref, gate_w_ref, gate_b_ref,
                w1_hbm, w2_hbm, b1a_ref, b1b_ref, b2_ref, psel_ref,
                out_ref, t_ref, r_ref, w1_buf, w2_buf, sems):
    e = pl.program_id(0)
    slot = jax.lax.rem(e, NSLOT)

    @pl.when(e == 0)
    def _warmup():
        for k in range(1):
            for c in _copies(w1_hbm, w2_hbm, w1_buf, w2_buf, sems, k, k):
                c.start()

    @pl.when(e + 1 < E)
    def _lookahead():
        s = jax.lax.rem(e + 1, NSLOT)
        for c in _copies(w1_hbm, w2_hbm, w1_buf, w2_buf, sems, e + 1, s):
            c.start()

    @pl.when(e == 0)
    def _prologue():
        x = x_ref[...]
        ms = jnp.mean(x * x, axis=1, keepdims=True)
        t = x * jax.lax.rsqrt(ms + EPS) * scale_ref[...]
        g = jax.lax.dot_general(
            t, gate_w_ref[...], (((1,), (1,)), ((), ())),
            preferred_element_type=jnp.float32) + gate_b_ref[...]
        t_ref[...] = t.astype(jnp.bfloat16)
        idx = jax.lax.broadcasted_iota(jnp.int32, (B, E), 1)
        v1 = jnp.max(g, axis=1, keepdims=True)
        i1 = jnp.min(jnp.where(g == v1, idx, E), axis=1, keepdims=True)
        g2 = jnp.where(idx == i1, NEG_BIG, g)
        v2 = jnp.max(g2, axis=1, keepdims=True)
        i2 = jnp.min(jnp.where(g2 == v2, idx, E), axis=1, keepdims=True)
        e2 = jnp.exp(v2 - v1)
        denom = 1.0 + e2
        r_ref[...] = (jnp.where(idx == i1, 1.0, 0.0)
                      + jnp.where(idx == i2, e2, 0.0)) / denom

    for c in _copies(w1_hbm, w2_hbm, w1_buf, w2_buf, sems, e, slot):
        c.wait()

    t16 = t_ref[...]
    h_a = _ffn_half(t16, w1_buf[slot, :I, :], b1a_ref, psel_ref)
    h_b = _ffn_half(t16, w1_buf[slot, I:, :], b1b_ref, psel_ref)
    w2 = w2_buf[slot]
    y = (jax.lax.dot_general(
            h_a, w2[:, :I // 2].astype(jnp.bfloat16), (((1,), (1,)), ((), ())),
            preferred_element_type=jnp.float32)
         + jax.lax.dot_general(
            h_b, w2[:, I // 2:].astype(jnp.bfloat16), (((1,), (1,)), ((), ())),
            preferred_element_type=jnp.float32)
         + b2_ref[...])
    lane = jax.lax.broadcasted_iota(jnp.int32, (B, E), 1)
    rw = jnp.sum(jnp.where(lane == e, r_ref[...], 0.0), axis=1, keepdims=True)
    contrib = y * rw

    @pl.when(e == 0)
    def _init():
        out_ref[...] = x_ref[...] + contrib

    @pl.when(e != 0)
    def _acc():
        out_ref[...] += contrib


def kernel(x, scale, gate_w, gate_b, mlp1_weight, mlp1_bias, mlp2_weight, mlp2_bias):
    b1a = mlp1_bias[:, :I].reshape(E, 1, I)
    b1b = mlp1_bias[:, I:].reshape(E, 1, I)
    b2 = mlp2_bias.reshape(E, 1, H)
    scale2 = scale.reshape(1, H)
    gate_b2 = gate_b.reshape(1, E)
    row = jax.lax.broadcasted_iota(jnp.int32, (I, I // 2), 0)
    col = jax.lax.broadcasted_iota(jnp.int32, (I, I // 2), 1)
    psel = (row == 2 * col).astype(jnp.bfloat16)

    return pl.pallas_call(
        _moe_kernel,
        grid=(E,),
        in_specs=[
            pl.BlockSpec((B, H), lambda e: (0, 0)),
            pl.BlockSpec((1, H), lambda e: (0, 0)),
            pl.BlockSpec((E, H), lambda e: (0, 0)),
            pl.BlockSpec((1, E), lambda e: (0, 0)),
            pl.BlockSpec(memory_space=pltpu.MemorySpace.HBM),
            pl.BlockSpec(memory_space=pltpu.MemorySpace.HBM),
            pl.BlockSpec((None, 1, I), lambda e: (e, 0, 0)),
            pl.BlockSpec((None, 1, I), lambda e: (e, 0, 0)),
            pl.BlockSpec((None, 1, H), lambda e: (e, 0, 0)),
            pl.BlockSpec((I, I // 2), lambda e: (0, 0)),
        ],
        out_specs=pl.BlockSpec((B, H), lambda e: (0, 0)),
        out_shape=jax.ShapeDtypeStruct((B, H), jnp.float32),
        scratch_shapes=[
            pltpu.VMEM((B, H), jnp.bfloat16),
            pltpu.VMEM((B, E), jnp.float32),
            pltpu.VMEM((NSLOT, 2 * I, H), jnp.float32),
            pltpu.VMEM((NSLOT, H, I), jnp.float32),
            pltpu.SemaphoreType.DMA((3, NSLOT)),
        ],
        compiler_params=pltpu.CompilerParams(
            dimension_semantics=("arbitrary",),
        ),
    )(x, scale2, gate_w, gate_b2, mlp1_weight, mlp2_weight, b1a, b1b, b2, psel)


# speedup vs baseline: 1.1443x; 1.1443x over previous
import jax
import jax.numpy as jnp
from jax.experimental import pallas as pl
from jax.experimental.pallas import tpu as pltpu

B = 32
H = 768
I = 768
E = 16
NSLOT = 3
ALPHA = 1.702
LIMIT = 7.0
EPS = 1e-5
NEG_BIG = -1e30


def _copies(w1_hbm, w2_hbm, w1_buf, w2_buf, sems, e, s):
    return (
        pltpu.make_async_copy(
            w1_hbm.at[pl.ds(e, 1), pl.ds(0, I), :],
            w1_buf.at[pl.ds(s, 1), pl.ds(0, I), :], sems.at[0, s]),
        pltpu.make_async_copy(
            w1_hbm.at[pl.ds(e, 1), pl.ds(I, I), :],
            w1_buf.at[pl.ds(s, 1), pl.ds(I, I), :], sems.at[1, s]),
        pltpu.make_async_copy(
            w2_hbm.at[pl.ds(e, 1)],
            w2_buf.at[pl.ds(s, 1)], sems.at[2, s]),
    )


def _ffn_half(t16, w1h, b1_ref, psel_ref):
    t1 = jax.lax.dot_general(
        t16, w1h.astype(jnp.bfloat16), (((1,), (1,)), ((), ())),
        preferred_element_type=jnp.float32) + b1_ref[...]
    glu = jnp.minimum(t1, LIMIT)
    og = glu * jax.nn.sigmoid(ALPHA * glu)
    lin = jnp.clip(t1, -LIMIT, LIMIT) + 1.0
    lin_next = pltpu.roll(lin, I - 1, 1)
    h_full = (og * lin_next).astype(jnp.bfloat16)
    return jax.lax.dot_general(
        h_full, psel_ref[...], (((1,), (0,)), ((), ())),
        preferred_element_type=jnp.float32).astype(jnp.bfloat16)


def _moe_kernel(x_ref, scale_ref, gate_w_ref, gate_b_ref,
                w1_hbm, w2_hbm, b1a_ref, b1b_ref, b2_ref, psel_ref,
                out_ref, t_ref, r_ref, w1_buf, w2_buf, sems):
    e = pl.program_id(0)
    slot = jax.lax.rem(e, NSLOT)

    @pl.when(e == 0)
    def _warmup():
        for k in range(2):
            for c in _copies(w1_hbm, w2_hbm, w1_buf, w2_buf, sems, k, k):
                c.start()

    @pl.when(e + 2 < E)
    def _lookahead():
        s = jax.lax.rem(e + 2, NSLOT)
        for c in _copies(w1_hbm, w2_hbm, w1_buf, w2_buf, sems, e + 2, s):
            c.start()

    @pl.when(e == 0)
    def _prologue():
        x = x_ref[...]
        ms = jnp.mean(x * x, axis=1, keepdims=True)
        t = x * jax.lax.rsqrt(ms + EPS) * scale_ref[...]
        g = jax.lax.dot_general(
            t, gate_w_ref[...], (((1,), (1,)), ((), ())),
            preferred_element_type=jnp.float32) + gate_b_ref[...]
        t_ref[...] = t.astype(jnp.bfloat16)
        idx = jax.lax.broadcasted_iota(jnp.int32, (B, E), 1)
        v1 = jnp.max(g, axis=1, keepdims=True)
        i1 = jnp.min(jnp.where(g == v1, idx, E), axis=1, keepdims=True)
        g2 = jnp.where(idx == i1, NEG_BIG, g)
        v2 = jnp.max(g2, axis=1, keepdims=True)
        i2 = jnp.min(jnp.where(g2 == v2, idx, E), axis=1, keepdims=True)
        e2 = jnp.exp(v2 - v1)
        denom = 1.0 + e2
        r_ref[...] = (jnp.where(idx == i1, 1.0, 0.0)
                      + jnp.where(idx == i2, e2, 0.0)) / denom

    for c in _copies(w1_hbm, w2_hbm, w1_buf, w2_buf, sems, e, slot):
        c.wait()

    t16 = t_ref[...]
    h_a = _ffn_half(t16, w1_buf[slot, :I, :], b1a_ref, psel_ref)
    h_b = _ffn_half(t16, w1_buf[slot, I:, :], b1b_ref, psel_ref)
    w2 = w2_buf[slot]
    y = (jax.lax.dot_general(
            h_a, w2[:, :I // 2].astype(jnp.bfloat16), (((1,), (1,)), ((), ())),
            preferred_element_type=jnp.float32)
         + jax.lax.dot_general(
            h_b, w2[:, I // 2:].astype(jnp.bfloat16), (((1,), (1,)), ((), ())),
            preferred_element_type=jnp.float32)
         + b2_ref[...])
    lane = jax.lax.broadcasted_iota(jnp.int32, (B, E), 1)
    rw = jnp.sum(jnp.where(lane == e, r_ref[...], 0.0), axis=1, keepdims=True)
    contrib = y * rw

    @pl.when(e == 0)
    def _init():
        out_ref[...] = x_ref[...] + contrib

    @pl.when(e != 0)
    def _acc():
        out_ref[...] += contrib


def kernel(x, scale, gate_w, gate_b, mlp1_weight, mlp1_bias, mlp2_weight, mlp2_bias):
    b1a = mlp1_bias[:, :I].reshape(E, 1, I)
    b1b = mlp1_bias[:, I:].reshape(E, 1, I)
    b2 = mlp2_bias.reshape(E, 1, H)
    scale2 = scale.reshape(1, H)
    gate_b2 = gate_b.reshape(1, E)
    row = jax.lax.broadcasted_iota(jnp.int32, (I, I // 2), 0)
    col = jax.lax.broadcasted_iota(jnp.int32, (I, I // 2), 1)
    psel = (row == 2 * col).astype(jnp.bfloat16)

    return pl.pallas_call(
        _moe_kernel,
        grid=(E,),
        in_specs=[
            pl.BlockSpec((B, H), lambda e: (0, 0)),
            pl.BlockSpec((1, H), lambda e: (0, 0)),
            pl.BlockSpec((E, H), lambda e: (0, 0)),
            pl.BlockSpec((1, E), lambda e: (0, 0)),
            pl.BlockSpec(memory_space=pltpu.MemorySpace.HBM),
            pl.BlockSpec(memory_space=pltpu.MemorySpace.HBM),
            pl.BlockSpec((None, 1, I), lambda e: (e, 0, 0)),
            pl.BlockSpec((None, 1, I), lambda e: (e, 0, 0)),
            pl.BlockSpec((None, 1, H), lambda e: (e, 0, 0)),
            pl.BlockSpec((I, I // 2), lambda e: (0, 0)),
        ],
        out_specs=pl.BlockSpec((B, H), lambda e: (0, 0)),
        out_shape=jax.ShapeDtypeStruct((B, H), jnp.float32),
        scratch_shapes=[
            pltpu.VMEM((B, H), jnp.bfloat16),
            pltpu.VMEM((B, E), jnp.float32),
            pltpu.VMEM((NSLOT, 2 * I, H), jnp.float32),
            pltpu.VMEM((NSLOT, H, I), jnp.float32),
            pltpu.SemaphoreType.DMA((3, NSLOT)),
        ],
        compiler_params=pltpu.CompilerParams(
            dimension_semantics=("arbitrary",),
        ),
    )(x, scale2, gate_w, gate_b2, mlp1_weight, mlp2_weight, b1a, b1b, b2, psel)
